# baseline (speedup 1.0000x reference)
import functools

import jax
import jax.numpy as jnp
from jax import lax
from jax.experimental import pallas as pl
from jax.experimental.pallas import tpu as pltpu
from jax.experimental.pallas import tpu_sc as plsc

VOCAB = 1000000
EMBED = 32
BATCH = 4096
SEQ = 200

ROW = 128
NROWS = BATCH * SEQ // ROW
NW = 32
ROWS_PER_W = NROWS // NW

NBLK = VOCAB // ROW
NTJ = NBLK + 1
VPAD = NTJ * ROW
LIN_ROWS = VOCAB * EMBED // ROW

_MESH = dict(core_axis_name="c", subcore_axis_name="s")
_LINEAR_PARAMS = pltpu.CompilerParams(
    use_tc_tiling_on_sc=False, needs_layout_passes=False
)


def _wid():
  return lax.axis_index("s") * 2 + lax.axis_index("c")


def _make_a1():
  CW = 8 * ROW
  NCH = 122

  @functools.partial(
      pl.kernel,
      mesh=plsc.VectorSubcoreMesh(**_MESH),
      compiler_params=pltpu.CompilerParams(use_tc_tiling_on_sc=True),
      out_type=jax.ShapeDtypeStruct((4, 8, VPAD), jnp.float32),
      scratch_types=(
          [pltpu.VMEM((8, CW), jnp.float32) for _ in range(8)]
          + [pltpu.SemaphoreType.DMA for _ in range(16)]
      ),
  )
  def body(tt_hbm, raw_hbm, *sc):
    bufs = sc[0:8]
    rs = sc[8:16]
    ws2 = sc[16:24]
    wid = _wid()
    et = wid // 8
    k = wid % 8
    W0 = 977 * ROW
    W1 = 976 * ROW
    x0 = jnp.where(k < 4, k * W0, 4 * W0 + (k - 4) * W1)

    def fire_in(t, b):
      off = x0 + t * CW
      pltpu.async_copy(tt_hbm.at[pl.ds(et * 8, 8), pl.ds(off, CW)],
                       bufs[b], rs[b])

    def wait_in(b):
      pltpu.make_async_copy(tt_hbm.at[pl.ds(0, 8), pl.ds(0, CW)],
                            bufs[b], rs[b]).wait()

    def fire_out(t, b):
      off = x0 + t * CW
      pltpu.async_copy(bufs[b], raw_hbm.at[et, :, pl.ds(off, CW)], ws2[b])

    def wait_out(b):
      pltpu.make_async_copy(bufs[b], raw_hbm.at[0, :, pl.ds(0, CW)],
                            ws2[b]).wait()

    for b in range(4):
      fire_in(b, b)

    def step(t, carry):
      for b in range(8):
        u = 8 * t + b

        @pl.when(u < NCH)
        def _(u=u, b=b):
          wait_in(b)
          fire_out(u, b)

          @pl.when(u >= 4)
          def _(u=u, b=b):
            wait_out((b + 4) % 8)

          @pl.when(u + 4 < NCH)
          def _(u=u, b=b):
            fire_in(u + 4, (b + 4) % 8)

      return carry

    lax.fori_loop(0, 16, step, 0)
    wait_out(6)
    wait_out(7)
    wait_out(0)
    wait_out(1)

    @pl.when(k < 4)
    def _rem():
      off = x0 + NCH * CW
      pltpu.sync_copy(tt_hbm.at[pl.ds(et * 8, 8), pl.ds(off, ROW)],
                      bufs[0].at[:, pl.ds(0, ROW)])
      pltpu.sync_copy(bufs[0].at[:, pl.ds(0, ROW)],
                      raw_hbm.at[et, :, pl.ds(off, ROW)])

  return body


def _make_a2():
  G = 4

  @functools.partial(
      pl.kernel,
      mesh=plsc.VectorSubcoreMesh(**_MESH),
      compiler_params=_LINEAR_PARAMS,
      out_type=jax.ShapeDtypeStruct((LIN_ROWS, ROW), jnp.float32),
      scratch_types=(
          [pltpu.VMEM((4 * G, 8, ROW), jnp.float32) for _ in range(2)]
          + [pltpu.VMEM((G * EMBED, ROW), jnp.float32) for _ in range(2)]
          + [pltpu.SemaphoreType.DMA for _ in range(4)]
      ),
  )
  def body(raw_hbm, tail_hbm, lin_hbm, *sc):
    ins = sc[0:2]
    outs = sc[2:4]
    gs = sc[4:6]
    ws2 = sc[6:8]
    wid = _wid()
    nq = 61 + (wid == 0)
    c0 = jnp.where(wid == 0, 0, 248 + (wid - 1) * 244)

    iota = lax.iota(jnp.int32, 16)
    e_vecs = [iota + h * 16 for h in range(2)]
    d1s = [e & 7 for e in e_vecs]
    d0s = [(e >> 3) * G for e in e_vecs]

    def fire_in(q, s):
      c = c0 + G * q
      for et in range(4):
        pltpu.async_copy(raw_hbm.at[pl.ds(et * NTJ + c, G)],
                         ins[s].at[pl.ds(et * G, G)], gs[s])

    def wait_in(s):
      for _ in range(4):
        pltpu.make_async_copy(raw_hbm.at[pl.ds(0, G)],
                              ins[s].at[pl.ds(0, G)], gs[s]).wait()

    def transpose(s):
      def blk(i, carry):
        d0i = [d0s[0] + i, d0s[1] + i]
        for j in range(256):
          half = j % 2
          v = plsc.load_gather(
              ins[s],
              [d0i[half], d1s[half], jnp.full((16,), j // 2, jnp.int32)],
          )
          flat = 16 * j
          outs[s][i * EMBED + flat // 128, pl.ds(flat % 128, 16)] = v
        return carry

      lax.fori_loop(0, G, blk, 0)

    def fire_out(q, s):
      pltpu.async_copy(outs[s],
                       lin_hbm.at[pl.ds((c0 + G * q) * EMBED, G * EMBED)],
                       ws2[s])

    def wait_out(s):
      pltpu.make_async_copy(outs[s], lin_hbm.at[pl.ds(0, G * EMBED)],
                            ws2[s]).wait()

    fire_in(0, 0)

    def step(q, carry):
      for s in range(2):

        @pl.when(q % 2 == s)
        def _(s=s):
          wait_in(s)

          @pl.when(q + 1 < nq)
          def _(s=s):
            fire_in(q + 1, 1 - s)

          @pl.when(q >= 2)
          def _(s=s):
            wait_out(s)

          transpose(s)
          fire_out(q, s)

      return carry

    lax.fori_loop(0, nq, step, 0)
    wait_out(0)
    wait_out(1)

    @pl.when(wid == 17)
    def _tail():
      pltpu.sync_copy(tail_hbm, lin_hbm.at[pl.ds(NBLK * EMBED, 16)])

  return body


def _make_phase_b():
  G = 10

  @functools.partial(
      pl.kernel,
      mesh=plsc.VectorSubcoreMesh(**_MESH),
      compiler_params=_LINEAR_PARAMS,
      out_type=jax.ShapeDtypeStruct((NROWS * 4, 8, ROW), jnp.float32),
      scratch_types=(
          [pltpu.VMEM((ROWS_PER_W, ROW), jnp.int32)]
          + [pltpu.VMEM((G * ROW, EMBED), jnp.float32) for _ in range(2)]
          + [pltpu.VMEM((4, 8, ROW), jnp.float32) for _ in range(2)]
          + [pltpu.SemaphoreType.DMA for _ in range(4)]
      ),
  )
  def body(idx_hbm, tab_hbm, out_hbm, idx_all, *sc):
    rows = sc[0:2]
    trs = sc[2:4]
    gs = sc[4:6]
    ws2 = sc[6:8]
    wid = _wid()
    q0 = wid * ROWS_PER_W
    pltpu.sync_copy(idx_hbm.at[pl.ds(q0, ROWS_PER_W)], idx_all)

    iota = lax.iota(jnp.int32, 16)
    b_vecs = [iota + h * 16 for h in range(8)]

    def fire_g(t, s):
      for g in range(G):
        pltpu.async_copy(tab_hbm.at[idx_all.at[G * t + g]],
                         rows[s].at[pl.ds(g * ROW, ROW)], gs[s])

    def wait_g(s):
      pltpu.make_async_copy(tab_hbm.at[pl.ds(0, G * ROW)], rows[s],
                            gs[s]).wait()

    def transpose(g, s, p):
      bg = [b + g * ROW for b in b_vecs]
      for j in range(256):
        v = plsc.load_gather(
            rows[s],
            [bg[j % 8], jnp.full((16,), j // 8, jnp.int32)],
        )
        flat = 16 * j
        trs[p][flat // 1024, (flat % 1024) // 128, pl.ds(flat % 128, 16)] = v

    def fire_w(n, p):
      q = q0 + n
      st = q // 256
      r = q % 256
      bt = r // 8
      si = r % 8
      s = st * 8 + si
      for et in range(4):
        pltpu.async_copy(trs[p].at[et], out_hbm.at[(s * 4 + et) * 32 + bt],
                         ws2[p])

    def wait_w(p):
      pltpu.make_async_copy(trs[p], out_hbm.at[pl.ds(0, 4)], ws2[p]).wait()

    fire_g(0, 0)

    def step(t, carry):
      for s in range(2):

        @pl.when(t % 2 == s)
        def _(s=s):
          wait_g(s)

          @pl.when(t + 1 < 20)
          def _(s=s):
            fire_g(t + 1, 1 - s)

          def inner(g, carry2):
            n = G * t + g
            for p in range(2):

              @pl.when(g % 2 == p)
              def _(p=p):

                @pl.when(n >= 2)
                def _(p=p):
                  wait_w(p)

                transpose(g, s, p)
                fire_w(n, p)

            return carry2

          lax.fori_loop(0, G, inner, 0)

      return carry

    lax.fori_loop(0, 20, step, 0)
    wait_w(0)
    wait_w(1)

  return body


_a1 = _make_a1()
_a2 = _make_a2()
_phase_b = _make_phase_b()


def kernel(words_seq, table):
  raw = _a1(table.T)
  raw3 = raw.reshape(4, 8, NTJ, ROW).transpose(0, 2, 1, 3).reshape(
      4 * NTJ, 8, ROW)
  tail = table[NBLK * ROW :, :].reshape(16, ROW)
  tab = _a2(raw3, tail).reshape(VOCAB, EMBED)
  ws = words_seq.astype(jnp.int32)
  idx = ws.T.reshape(25, 8, 32, 128).transpose(0, 2, 1, 3).reshape(NROWS, ROW)
  out = _phase_b(idx, tab)
  out5 = out.reshape(SEQ, 4, 32, 8, ROW)
  return out5.transpose(2, 4, 0, 1, 3).reshape(BATCH, SEQ, EMBED)


# speedup vs baseline: 1.6179x; 1.6179x over previous
import functools

import jax
import jax.numpy as jnp
from jax import lax
from jax.experimental import pallas as pl
from jax.experimental.pallas import tpu as pltpu
from jax.experimental.pallas import tpu_sc as plsc

VOCAB = 1000000
EMBED = 32
BATCH = 4096
SEQ = 200

ROW = 128
NROWS = BATCH * SEQ // ROW
NW = 32
ROWS_PER_W = NROWS // NW
K = 10
NGROUPS = ROWS_PER_W // K


def _make_kernel():
  mesh = plsc.VectorSubcoreMesh(core_axis_name="c", subcore_axis_name="s")

  @functools.partial(
      pl.kernel,
      mesh=mesh,
      compiler_params=pltpu.CompilerParams(use_tc_tiling_on_sc=False),
      out_type=jax.ShapeDtypeStruct((NROWS, ROW, EMBED), jnp.float32),
      scratch_types=[
          pltpu.VMEM((ROWS_PER_W, ROW), jnp.int32),
          pltpu.VMEM((K, ROW, EMBED), jnp.float32),
          pltpu.VMEM((K, ROW, EMBED), jnp.float32),
          pltpu.SemaphoreType.DMA,
          pltpu.SemaphoreType.DMA,
          pltpu.SemaphoreType.DMA,
          pltpu.SemaphoreType.DMA,
      ],
  )
  def body(idx_hbm, table_hbm, out_hbm, idx_all, rows0, rows1,
           gsem0, gsem1, wsem0, wsem1):
    wid = lax.axis_index("s") * 2 + lax.axis_index("c")
    base = wid * ROWS_PER_W

    pltpu.sync_copy(idx_hbm.at[pl.ds(base, ROWS_PER_W)], idx_all)

    def fire(g, rows, gsem):
      for j in range(K):
        pltpu.async_copy(table_hbm.at[idx_all.at[g * K + j]], rows.at[j], gsem)

    def drain(g, rows, gsem):
      pltpu.make_async_copy(out_hbm.at[pl.ds(0, K)], rows, gsem).wait()

    def wb(g, rows, wsem):
      pltpu.async_copy(rows, out_hbm.at[pl.ds(base + g * K, K)], wsem)

    def wait_wb(g, rows, wsem):
      pltpu.make_async_copy(rows, out_hbm.at[pl.ds(base + g * K, K)], wsem).wait()

    fire(0, rows0, gsem0)
    fire(1, rows1, gsem1)
    drain(0, rows0, gsem0)
    wb(0, rows0, wsem0)

    def step(m, carry):
      g0 = 2 * m
      g1 = 2 * m + 1
      drain(g1 - 2, rows1, gsem1)
      wb(g1 - 2, rows1, wsem1)
      wait_wb(g0 - 2, rows0, wsem0)
      fire(g0, rows0, gsem0)
      drain(g0, rows0, gsem0)
      wb(g0, rows0, wsem0)
      wait_wb(g1 - 2, rows1, wsem1)
      fire(g1, rows1, gsem1)
      return carry

    lax.fori_loop(1, NGROUPS // 2, step, 0)

    drain(NGROUPS - 1, rows1, gsem1)
    wb(NGROUPS - 1, rows1, wsem1)
    wait_wb(NGROUPS - 2, rows0, wsem0)
    wait_wb(NGROUPS - 1, rows1, wsem1)

  return body


_sc_gather = _make_kernel()


def kernel(words_seq, table):
  idx = words_seq.reshape(NROWS, ROW).astype(jnp.int32)
  out = _sc_gather(idx, table)
  return out.reshape(BATCH, SEQ, EMBED)
